# baseline (speedup 1.0000x reference)
import jax
import jax.numpy as jnp
from jax.experimental import pallas as pl
from jax.experimental.pallas import tpu as pltpu

_BM = 400
_NBUF = 3
_TBM = 80
_NMAIN = 24
_NTAIL = 5


def _gcn_body(w_ref, bias_ref, x_hbm, adj_hbm, out_hbm,
              buf, sem, osb, wsem, x_vmem, xsem, sup_ref):
    pltpu.make_async_copy(x_hbm, x_vmem, xsem).start()
    for j in range(_NBUF):
        pltpu.make_async_copy(
            adj_hbm.at[pl.ds(j * _BM, _BM), :], buf.at[j], sem.at[j]
        ).start()
    pltpu.make_async_copy(x_hbm, x_vmem, xsem).wait()

    sup_ref[...] = jnp.dot(
        x_vmem[...].astype(jnp.bfloat16),
        w_ref[...].astype(jnp.bfloat16),
        preferred_element_type=jnp.float32,
    ).astype(jnp.bfloat16)

    def main_step(i, carry):
        slot = jax.lax.rem(i, _NBUF)
        pltpu.make_async_copy(
            adj_hbm.at[pl.ds(i * _BM, _BM), :], buf.at[slot], sem.at[slot]
        ).wait()

        @pl.when(i >= _NBUF)
        def _():
            prev = i - _NBUF
            pltpu.make_async_copy(
                osb.at[slot],
                out_hbm.at[pl.ds(prev * _BM, _BM), :],
                wsem.at[slot],
            ).wait()

        osb[slot] = (
            jnp.dot(
                buf[slot].astype(jnp.bfloat16),
                sup_ref[...],
                preferred_element_type=jnp.float32,
            )
            + bias_ref[...]
        )
        pltpu.make_async_copy(
            osb.at[slot], out_hbm.at[pl.ds(i * _BM, _BM), :], wsem.at[slot]
        ).start()

        nxt = i + _NBUF

        @pl.when(nxt < _NMAIN)
        def _():
            pltpu.make_async_copy(
                adj_hbm.at[pl.ds(nxt * _BM, _BM), :], buf.at[slot], sem.at[slot]
            ).start()

        @pl.when(jnp.logical_and(nxt >= _NMAIN, nxt < _NMAIN + _NTAIL))
        def _():
            t = nxt - _NMAIN
            pltpu.make_async_copy(
                adj_hbm.at[pl.ds(_NMAIN * _BM + t * _TBM, _TBM), :],
                buf.at[slot, 0:_TBM, :],
                sem.at[slot],
            ).start()

        return carry

    jax.lax.fori_loop(0, _NMAIN, main_step, 0)

    for t in range(_NTAIL):
        i = _NMAIN + t
        slot = i % _NBUF
        row = _NMAIN * _BM + t * _TBM
        pltpu.make_async_copy(
            adj_hbm.at[pl.ds(row, _TBM), :], buf.at[slot, 0:_TBM, :],
            sem.at[slot],
        ).wait()
        prev = i - _NBUF
        if prev < _NMAIN:
            pltpu.make_async_copy(
                osb.at[slot], out_hbm.at[pl.ds(prev * _BM, _BM), :],
                wsem.at[slot],
            ).wait()
        else:
            prow = _NMAIN * _BM + (prev - _NMAIN) * _TBM
            pltpu.make_async_copy(
                osb.at[slot, 0:_TBM, :], out_hbm.at[pl.ds(prow, _TBM), :],
                wsem.at[slot],
            ).wait()
        osb[slot, 0:_TBM, :] = (
            jnp.dot(
                buf[slot, 0:_TBM, :].astype(jnp.bfloat16),
                sup_ref[...],
                preferred_element_type=jnp.float32,
            )
            + bias_ref[...]
        )
        pltpu.make_async_copy(
            osb.at[slot, 0:_TBM, :], out_hbm.at[pl.ds(row, _TBM), :],
            wsem.at[slot],
        ).start()
        nxt = i + _NBUF
        if nxt < _NMAIN + _NTAIL:
            nrow = _NMAIN * _BM + (nxt - _NMAIN) * _TBM
            pltpu.make_async_copy(
                adj_hbm.at[pl.ds(nrow, _TBM), :],
                buf.at[nxt % _NBUF, 0:_TBM, :],
                sem.at[nxt % _NBUF],
            ).start()

    for t in range(_NTAIL - _NBUF, _NTAIL):
        i = _NMAIN + t
        slot = i % _NBUF
        row = _NMAIN * _BM + t * _TBM
        pltpu.make_async_copy(
            osb.at[slot, 0:_TBM, :], out_hbm.at[pl.ds(row, _TBM), :],
            wsem.at[slot],
        ).wait()


def kernel(x, adj, weight, bias):
    n, d_in = x.shape
    d_out = weight.shape[1]
    return pl.pallas_call(
        _gcn_body,
        in_specs=[
            pl.BlockSpec(memory_space=pltpu.MemorySpace.VMEM),
            pl.BlockSpec(memory_space=pltpu.MemorySpace.VMEM),
            pl.BlockSpec(memory_space=pltpu.MemorySpace.HBM),
            pl.BlockSpec(memory_space=pltpu.MemorySpace.HBM),
        ],
        out_specs=pl.BlockSpec(memory_space=pltpu.MemorySpace.HBM),
        out_shape=jax.ShapeDtypeStruct((n, d_out), x.dtype),
        scratch_shapes=[
            pltpu.VMEM((_NBUF, _BM, n), jnp.float32),
            pltpu.SemaphoreType.DMA((_NBUF,)),
            pltpu.VMEM((_NBUF, _BM, 128), jnp.float32),
            pltpu.SemaphoreType.DMA((_NBUF,)),
            pltpu.VMEM((n, d_in), jnp.float32),
            pltpu.SemaphoreType.DMA,
            pltpu.VMEM((n, d_out), jnp.bfloat16),
        ],
    )(weight, bias.reshape(1, d_out), x, adj)


# speedup vs baseline: 1.0249x; 1.0249x over previous
import jax
import jax.numpy as jnp
from jax.experimental import pallas as pl
from jax.experimental.pallas import tpu as pltpu

_BM = 200
_S = 2


def _gcn_body(x_ref, w_ref, *rest):
    adj_refs = rest[:_S]
    bias_ref = rest[_S]
    out_ref = rest[_S + 1]
    sup_ref = rest[_S + 2]

    @pl.when(pl.program_id(0) == 0)
    def _():
        sup_ref[...] = jnp.dot(
            x_ref[...].astype(jnp.bfloat16),
            w_ref[...].astype(jnp.bfloat16),
            preferred_element_type=jnp.float32,
        ).astype(jnp.bfloat16)

    for j in range(_S):
        out_ref[j * _BM:(j + 1) * _BM, :] = (
            jnp.dot(
                adj_refs[j][...].astype(jnp.bfloat16),
                sup_ref[...],
                preferred_element_type=jnp.float32,
            )
            + bias_ref[...]
        )


def kernel(x, adj, weight, bias):
    n, d_in = x.shape
    d_out = weight.shape[1]
    bm, s = _BM, _S
    rows_per_step = s * bm
    adj_specs = [
        pl.BlockSpec((bm, n), lambda i, j=j: (i * s + j, 0)) for j in range(s)
    ]
    return pl.pallas_call(
        _gcn_body,
        grid=(n // rows_per_step,),
        in_specs=[
            pl.BlockSpec((n, d_in), lambda i: (0, 0)),
            pl.BlockSpec((d_in, d_out), lambda i: (0, 0)),
            *adj_specs,
            pl.BlockSpec((1, d_out), lambda i: (0, 0)),
        ],
        out_specs=pl.BlockSpec((rows_per_step, d_out), lambda i: (i, 0)),
        out_shape=jax.ShapeDtypeStruct((n, d_out), x.dtype),
        scratch_shapes=[pltpu.VMEM((n, d_out), jnp.bfloat16)],
        compiler_params=pltpu.CompilerParams(
            dimension_semantics=("arbitrary",)
        ),
    )(x, weight, *([adj] * s), bias.reshape(1, d_out))
